# baseline (speedup 1.0000x reference)
import functools

import jax
import jax.numpy as jnp
from jax import lax
from jax.experimental import pallas as pl
from jax.experimental.pallas import tpu as pltpu
from jax.experimental.pallas import tpu_sc as plsc

_B = 16384
_F = 64
_NC = 2
_NS = 16
_NW = _NC * _NS
_BPW = _B // _NW
_L = 16
_IDXC = 128
_NK = _BPW // _IDXC
_NT = _IDXC // _L

_OFF_SH = 24
_OFF_U = 88
_OFF_I = 152
_OFF_B = 216
_WBLEN = 240


def _body(uid_hbm, sid_hbm, iid_hbm, info_hbm, vis_hbm, buy_hbm,
          ut_hbm, it_hbm, st_hbm, wb_hbm, out_hbm,
          uid_v, sid_v, iid_v, info_v, vis_v, buy_v,
          urows_v, irows_v, stab_v, sprj_v, wb_v, out_v,
          sems_u, sems_i):
    wid = lax.axis_index("s") * _NC + lax.axis_index("c")
    base = wid * _BPW

    pltpu.sync_copy(uid_hbm.at[pl.ds(wid * _NK, _NK)], uid_v)
    pltpu.sync_copy(iid_hbm.at[pl.ds(wid * _NK, _NK)], iid_v)

    cps_u = []
    cps_i = []
    for k in range(_NK):
        cps_u.append(pltpu.async_copy(
            ut_hbm.at[uid_v.at[k]], urows_v.at[pl.ds(k * _IDXC, _IDXC)],
            sems_u.at[k]))
        cps_i.append(pltpu.async_copy(
            it_hbm.at[iid_v.at[k]], irows_v.at[pl.ds(k * _IDXC, _IDXC)],
            sems_i.at[k]))

    pltpu.sync_copy(sid_hbm.at[pl.ds(base, _BPW)], sid_v)
    pltpu.sync_copy(info_hbm.at[pl.ds(base, _BPW)], info_v)
    pltpu.sync_copy(vis_hbm.at[pl.ds(base, _BPW)], vis_v)
    pltpu.sync_copy(buy_hbm.at[pl.ds(base, _BPW)], buy_v)
    pltpu.sync_copy(wb_hbm, wb_v)
    pltpu.sync_copy(st_hbm, stab_v)

    lane = lax.iota(jnp.int32, _L)
    wsh = [wb_v[pl.ds(_OFF_SH + c * _L, _L)] for c in range(_F // _L)]
    sprj = jnp.zeros((_L,), jnp.float32)
    for j in range(_F):
        jv = jnp.full((_L,), j, jnp.int32)
        sprj = sprj + plsc.load_gather(stab_v, [lane, jv]) * wsh[j // _L][j % _L]
    sprj_v[...] = sprj

    wctx = [wb_v[pl.ds(0, _L)], wb_v[pl.ds(_L, _L)]]
    wu = [wb_v[pl.ds(_OFF_U + c * _L, _L)] for c in range(_F // _L)]
    wi = [wb_v[pl.ds(_OFF_I + c * _L, _L)] for c in range(_F // _L)]
    bias = wb_v[pl.ds(_OFF_B, _L)][0]

    def chunk_body(t, carry):
        off = t * _L
        ivec = off + lax.iota(jnp.int32, _L)
        acc = jnp.full((_L,), 0.0, jnp.float32) + bias
        for j in range(22):
            jv = jnp.full((_L,), j, jnp.int32)
            acc = acc + plsc.load_gather(info_v, [ivec, jv]) * wctx[j // _L][j % _L]
        acc = acc + vis_v[pl.ds(off, _L)] * wctx[1][6]
        acc = acc + buy_v[pl.ds(off, _L)] * wctx[1][7]
        acc = acc + plsc.load_gather(sprj_v, [sid_v[pl.ds(off, _L)]])
        for j in range(_F):
            jv = jnp.full((_L,), j, jnp.int32)
            acc = acc + plsc.load_gather(urows_v, [ivec, jv]) * wu[j // _L][j % _L]
            acc = acc + plsc.load_gather(irows_v, [ivec, jv]) * wi[j // _L][j % _L]
        out_v[pl.ds(off, _L)] = 1.0 / (1.0 + jnp.exp(-acc))
        return carry

    for k in range(_NK):
        cps_u[k].wait()
        cps_i[k].wait()
        lax.fori_loop(k * _NT, (k + 1) * _NT, chunk_body, jnp.int32(0))

    pltpu.sync_copy(out_v, out_hbm.at[pl.ds(base, _BPW)])


@functools.cache
def _get_sc_call():
  return pl.kernel(
    _body,
    out_type=jax.ShapeDtypeStruct((_B,), jnp.float32),
    mesh=plsc.VectorSubcoreMesh(core_axis_name="c", subcore_axis_name="s",
                                num_cores=_NC, num_subcores=_NS),
    compiler_params=pltpu.CompilerParams(needs_layout_passes=False,
                                         use_tc_tiling_on_sc=False),
    scratch_types=[
        pltpu.VMEM((_NK, _IDXC), jnp.int32),
        pltpu.VMEM((_BPW,), jnp.int32),
        pltpu.VMEM((_NK, _IDXC), jnp.int32),
        pltpu.VMEM((_BPW, 22), jnp.float32),
        pltpu.VMEM((_BPW,), jnp.float32),
        pltpu.VMEM((_BPW,), jnp.float32),
        pltpu.VMEM((_BPW, _F), jnp.float32),
        pltpu.VMEM((_BPW, _F), jnp.float32),
        pltpu.VMEM((_L, _F), jnp.float32),
        pltpu.VMEM((_L,), jnp.float32),
        pltpu.VMEM((_WBLEN,), jnp.float32),
        pltpu.VMEM((_BPW,), jnp.float32),
        pltpu.SemaphoreType.DMA((_NK,)),
        pltpu.SemaphoreType.DMA((_NK,)),
    ],
  )


@jax.jit
def kernel(user_ids, shift_ids, item_ids, category, info, visits, buys,
           user_table, item_table, shift_table, W, b):
    del category
    uid = user_ids.astype(jnp.int32).reshape(_B // _IDXC, _IDXC)
    iid = item_ids.astype(jnp.int32).reshape(_B // _IDXC, _IDXC)
    sid = shift_ids.astype(jnp.int32)
    wb = jnp.concatenate([W.reshape(-1), b.astype(jnp.float32),
                          jnp.zeros((_WBLEN - _OFF_B - 1,), jnp.float32)])
    st_pad = jnp.zeros((_L, _F), jnp.float32).at[:10].set(shift_table)
    out = _get_sc_call()(uid, sid, iid, info, visits, buys,
                         user_table, item_table, st_pad, wb)
    return out.reshape(_B, 1)


# speedup vs baseline: 1.6123x; 1.6123x over previous
import functools

import jax
import jax.numpy as jnp
from jax import lax
from jax.experimental import pallas as pl
from jax.experimental.pallas import tpu as pltpu
from jax.experimental.pallas import tpu_sc as plsc

_B = 16384
_F = 64
_NC = 2
_NS = 16
_NW = _NC * _NS
_BPW = _B // _NW
_L = 16
_HB = _BPW // 2
_HT = _HB // _L
_NINFO = 22

_OFF_SH = 24
_OFF_U = 88
_OFF_I = 152
_OFF_B = 216
_WBLEN = 240


def _fire_row_dmas(table_hbm, ids_v, id_off, dst_v, sem):
    def grp(g, carry):
        idv = ids_v[pl.ds(id_off + g * _L, _L)]
        for k in range(_L):
            pltpu.async_copy(
                table_hbm.at[pl.ds(idv[k], 1)],
                dst_v.at[pl.ds(g * _L + k, 1)],
                sem)
        return carry
    lax.fori_loop(0, _HB // _L, grp, jnp.int32(0))


def _body(uid_hbm, sid_hbm, iid_hbm, info_hbm, vis_hbm, buy_hbm,
          ut_hbm, it_hbm, st_hbm, wb_hbm, out_hbm,
          uid_v, sid_v, iid_v, info_v, vis_v, buy_v,
          rowsA_v, rowsB_v, stab_v, sprj_v, wb_v, out_v,
          sem_a, sem_b):
    wid = lax.axis_index("s") * _NC + lax.axis_index("c")
    base = wid * _BPW

    pltpu.sync_copy(uid_hbm.at[pl.ds(base, _BPW)], uid_v)
    pltpu.sync_copy(iid_hbm.at[pl.ds(base, _BPW)], iid_v)
    _fire_row_dmas(ut_hbm, uid_v, 0, rowsA_v, sem_a)
    _fire_row_dmas(ut_hbm, uid_v, _HB, rowsB_v, sem_b)

    pltpu.sync_copy(sid_hbm.at[pl.ds(base, _BPW)], sid_v)
    pltpu.sync_copy(info_hbm.at[pl.ds(base * _NINFO, _BPW * _NINFO)], info_v)
    pltpu.sync_copy(vis_hbm.at[pl.ds(base, _BPW)], vis_v)
    pltpu.sync_copy(buy_hbm.at[pl.ds(base, _BPW)], buy_v)
    pltpu.sync_copy(wb_hbm, wb_v)
    pltpu.sync_copy(st_hbm, stab_v)

    wctx = [wb_v[pl.ds(0, _L)], wb_v[pl.ds(_L, _L)]]
    wsh = [wb_v[pl.ds(_OFF_SH + c * _L, _L)] for c in range(_F // _L)]
    wu = [wb_v[pl.ds(_OFF_U + c * _L, _L)] for c in range(_F // _L)]
    wi = [wb_v[pl.ds(_OFF_I + c * _L, _L)] for c in range(_F // _L)]
    bias = wb_v[pl.ds(_OFF_B, _L)][0]

    lane = lax.iota(jnp.int32, _L)
    sprj = jnp.zeros((_L,), jnp.float32)
    for j in range(_F):
        jv = jnp.full((_L,), j, jnp.int32)
        sprj = sprj + plsc.load_gather(stab_v, [lane, jv]) * wsh[j // _L][j % _L]
    sprj_v[...] = sprj

    def ctx_body(t, carry):
        off = t * _L
        ibase = (off + lax.iota(jnp.int32, _L)) * _NINFO
        a0 = jnp.full((_L,), 0.0, jnp.float32) + bias
        a1 = vis_v[pl.ds(off, _L)] * wctx[1][6]
        a2 = buy_v[pl.ds(off, _L)] * wctx[1][7]
        a3 = plsc.load_gather(sprj_v, [sid_v[pl.ds(off, _L)]])
        accs = [a0, a1, a2, a3]
        for j in range(_NINFO):
            accs[j % 4] = accs[j % 4] + (plsc.load_gather(info_v, [ibase + j])
                                         * wctx[j // _L][j % _L])
        out_v[pl.ds(off, _L)] = (accs[0] + accs[1]) + (accs[2] + accs[3])
        return carry

    lax.fori_loop(0, _BPW // _L, ctx_body, jnp.int32(0))

    def make_rows_pass(rows_v, w, t_off, last):
        def rows_body(t, carry):
            ivec = t * _L + lax.iota(jnp.int32, _L)
            a = [jnp.zeros((_L,), jnp.float32) for _ in range(4)]
            for j in range(_F):
                jv = jnp.full((_L,), j, jnp.int32)
                a[j % 4] = a[j % 4] + (plsc.load_gather(rows_v, [ivec, jv])
                                       * w[j // _L][j % _L])
            acc = (a[0] + a[1]) + (a[2] + a[3])
            off = t_off + t * _L
            if last:
                z = out_v[pl.ds(off, _L)] + acc
                out_v[pl.ds(off, _L)] = 1.0 / (1.0 + jnp.exp(-z))
            else:
                plsc.addupdate(out_v.at[pl.ds(off, _L)], acc)
            return carry
        return rows_body

    def drain(table_hbm, dst_v, sem):
        pltpu.make_async_copy(table_hbm.at[pl.ds(0, _HB)], dst_v, sem).wait()

    drain(ut_hbm, rowsA_v, sem_a)
    lax.fori_loop(0, _HT, make_rows_pass(rowsA_v, wu, 0, False), jnp.int32(0))
    _fire_row_dmas(it_hbm, iid_v, 0, rowsA_v, sem_a)

    drain(ut_hbm, rowsB_v, sem_b)
    lax.fori_loop(0, _HT, make_rows_pass(rowsB_v, wu, _HB, False), jnp.int32(0))
    _fire_row_dmas(it_hbm, iid_v, _HB, rowsB_v, sem_b)

    drain(it_hbm, rowsA_v, sem_a)
    lax.fori_loop(0, _HT, make_rows_pass(rowsA_v, wi, 0, True), jnp.int32(0))

    drain(it_hbm, rowsB_v, sem_b)
    lax.fori_loop(0, _HT, make_rows_pass(rowsB_v, wi, _HB, True), jnp.int32(0))

    pltpu.sync_copy(out_v, out_hbm.at[pl.ds(base, _BPW)])


@functools.cache
def _get_sc_call():
  return pl.kernel(
    _body,
    out_type=jax.ShapeDtypeStruct((_B,), jnp.float32),
    mesh=plsc.VectorSubcoreMesh(core_axis_name="c", subcore_axis_name="s",
                                num_cores=_NC, num_subcores=_NS),
    compiler_params=pltpu.CompilerParams(needs_layout_passes=False,
                                         use_tc_tiling_on_sc=True),
    scratch_types=[
        pltpu.VMEM((_BPW,), jnp.int32),
        pltpu.VMEM((_BPW,), jnp.int32),
        pltpu.VMEM((_BPW,), jnp.int32),
        pltpu.VMEM((_BPW * _NINFO,), jnp.float32),
        pltpu.VMEM((_BPW,), jnp.float32),
        pltpu.VMEM((_BPW,), jnp.float32),
        pltpu.VMEM((_HB, _F), jnp.float32),
        pltpu.VMEM((_HB, _F), jnp.float32),
        pltpu.VMEM((_L, _F), jnp.float32),
        pltpu.VMEM((_L,), jnp.float32),
        pltpu.VMEM((_WBLEN,), jnp.float32),
        pltpu.VMEM((_BPW,), jnp.float32),
        pltpu.SemaphoreType.DMA,
        pltpu.SemaphoreType.DMA,
    ],
  )


@jax.jit
def kernel(user_ids, shift_ids, item_ids, category, info, visits, buys,
           user_table, item_table, shift_table, W, b):
    del category
    uid = user_ids.astype(jnp.int32)
    iid = item_ids.astype(jnp.int32)
    sid = shift_ids.astype(jnp.int32)
    wb = jnp.concatenate([W.reshape(-1), b.astype(jnp.float32),
                          jnp.zeros((_WBLEN - _OFF_B - 1,), jnp.float32)])
    st_pad = jnp.zeros((_L, _F), jnp.float32).at[:10].set(shift_table)
    info_flat = info.reshape(-1)
    out = _get_sc_call()(uid, sid, iid, info_flat, visits, buys,
                         user_table, item_table, st_pad, wb)
    return out.reshape(_B, 1)


# speedup vs baseline: 1.6151x; 1.0017x over previous
import functools

import jax
import jax.numpy as jnp
from jax import lax
from jax.experimental import pallas as pl
from jax.experimental.pallas import tpu as pltpu
from jax.experimental.pallas import tpu_sc as plsc

_B = 16384
_F = 64
_NC = 2
_NS = 16
_NW = _NC * _NS
_BPW = _B // _NW
_L = 16
_HB = _BPW // 2
_HT = _HB // _L
_IDXC = 128
_NK = _BPW // _IDXC
_NINFO = 22

_OFF_SH = 24
_OFF_U = 88
_OFF_I = 152
_OFF_B = 216
_WBLEN = 240


def _fire_row_dmas(table_hbm, ids_v, id_off, dst_v, sem):
    def grp(g, carry):
        idv = ids_v[pl.ds(id_off + g * _L, _L)]
        for k in range(_L):
            pltpu.async_copy(
                table_hbm.at[pl.ds(idv[k], 1)],
                dst_v.at[pl.ds(g * _L + k, 1)],
                sem)
        return carry
    lax.fori_loop(0, _HB // _L, grp, jnp.int32(0))


def _user_body(uid_hbm, ut_hbm, wu_hbm, out_hbm,
               uid_v, rowsA_v, rowsB_v, wv_v, out_v, sem_a, sem_b):
    wid = lax.axis_index("s") * _NC + lax.axis_index("c")
    base = wid * _BPW

    pltpu.sync_copy(uid_hbm.at[pl.ds(base, _BPW)], uid_v)
    _fire_row_dmas(ut_hbm, uid_v, 0, rowsA_v, sem_a)
    _fire_row_dmas(ut_hbm, uid_v, _HB, rowsB_v, sem_b)
    pltpu.sync_copy(wu_hbm, wv_v)
    w = [wv_v[pl.ds(c * _L, _L)] for c in range(_F // _L)]

    def make_pass(rows_v, t_off):
        def body(t, carry):
            ivec = t * _L + lax.iota(jnp.int32, _L)
            a = [jnp.zeros((_L,), jnp.float32) for _ in range(4)]
            for j in range(_F):
                jv = jnp.full((_L,), j, jnp.int32)
                a[j % 4] = a[j % 4] + (plsc.load_gather(rows_v, [ivec, jv])
                                       * w[j // _L][j % _L])
            out_v[pl.ds(t_off + t * _L, _L)] = (a[0] + a[1]) + (a[2] + a[3])
            return carry
        return body

    def drain(dst_v, sem):
        pltpu.make_async_copy(ut_hbm.at[pl.ds(0, _HB)], dst_v, sem).wait()

    drain(rowsA_v, sem_a)
    lax.fori_loop(0, _HT, make_pass(rowsA_v, 0), jnp.int32(0))
    drain(rowsB_v, sem_b)
    lax.fori_loop(0, _HT, make_pass(rowsB_v, _HB), jnp.int32(0))

    pltpu.sync_copy(out_v, out_hbm.at[pl.ds(base, _BPW)])


@functools.cache
def _get_user_call():
  return pl.kernel(
    _user_body,
    out_type=jax.ShapeDtypeStruct((_B,), jnp.float32),
    mesh=plsc.VectorSubcoreMesh(core_axis_name="c", subcore_axis_name="s",
                                num_cores=_NC, num_subcores=_NS),
    compiler_params=pltpu.CompilerParams(needs_layout_passes=False,
                                         use_tc_tiling_on_sc=True),
    scratch_types=[
        pltpu.VMEM((_BPW,), jnp.int32),
        pltpu.VMEM((_HB, _F), jnp.float32),
        pltpu.VMEM((_HB, _F), jnp.float32),
        pltpu.VMEM((_F,), jnp.float32),
        pltpu.VMEM((_BPW,), jnp.float32),
        pltpu.SemaphoreType.DMA,
        pltpu.SemaphoreType.DMA,
    ],
  )


def _rest_body(iid_hbm, sid_hbm, info_hbm, vis_hbm, buy_hbm,
               it_hbm, st_hbm, wb_hbm, up_hbm, out_hbm,
               iid_v, sid_v, info_v, vis_v, buy_v,
               irows_v, stab_v, sprj_v, wb_v, up_v, out_v, sems):
    wid = lax.axis_index("s") * _NC + lax.axis_index("c")
    base = wid * _BPW

    pltpu.sync_copy(iid_hbm.at[pl.ds(wid * _NK, _NK)], iid_v)
    cps = []
    for k in range(_NK):
        cps.append(pltpu.async_copy(
            it_hbm.at[iid_v.at[k]],
            irows_v.at[pl.ds(k * _IDXC, _IDXC)],
            sems.at[k]))

    pltpu.sync_copy(sid_hbm.at[pl.ds(base, _BPW)], sid_v)
    pltpu.sync_copy(info_hbm.at[pl.ds(base * _NINFO, _BPW * _NINFO)], info_v)
    pltpu.sync_copy(vis_hbm.at[pl.ds(base, _BPW)], vis_v)
    pltpu.sync_copy(buy_hbm.at[pl.ds(base, _BPW)], buy_v)
    pltpu.sync_copy(up_hbm.at[pl.ds(base, _BPW)], up_v)
    pltpu.sync_copy(wb_hbm, wb_v)
    pltpu.sync_copy(st_hbm, stab_v)

    wctx = [wb_v[pl.ds(0, _L)], wb_v[pl.ds(_L, _L)]]
    wsh = [wb_v[pl.ds(_OFF_SH + c * _L, _L)] for c in range(_F // _L)]
    wi = [wb_v[pl.ds(_OFF_I + c * _L, _L)] for c in range(_F // _L)]
    bias = wb_v[pl.ds(_OFF_B, _L)][0]

    lane = lax.iota(jnp.int32, _L)
    sprj = jnp.zeros((_L,), jnp.float32)
    for j in range(_F):
        jv = jnp.full((_L,), j, jnp.int32)
        sprj = sprj + plsc.load_gather(stab_v, [lane, jv]) * wsh[j // _L][j % _L]
    sprj_v[...] = sprj

    def ctx_body(t, carry):
        off = t * _L
        ibase = (off + lax.iota(jnp.int32, _L)) * _NINFO
        a0 = up_v[pl.ds(off, _L)] + bias
        a1 = vis_v[pl.ds(off, _L)] * wctx[1][6]
        a2 = buy_v[pl.ds(off, _L)] * wctx[1][7]
        a3 = plsc.load_gather(sprj_v, [sid_v[pl.ds(off, _L)]])
        accs = [a0, a1, a2, a3]
        for j in range(_NINFO):
            accs[j % 4] = accs[j % 4] + (plsc.load_gather(info_v, [ibase + j])
                                         * wctx[j // _L][j % _L])
        out_v[pl.ds(off, _L)] = (accs[0] + accs[1]) + (accs[2] + accs[3])
        return carry

    lax.fori_loop(0, _BPW // _L, ctx_body, jnp.int32(0))

    def make_item_pass(t0):
        def body(t, carry):
            off = t * _L
            ivec = off + lax.iota(jnp.int32, _L)
            a = [jnp.zeros((_L,), jnp.float32) for _ in range(4)]
            for j in range(_F):
                jv = jnp.full((_L,), j, jnp.int32)
                a[j % 4] = a[j % 4] + (plsc.load_gather(irows_v, [ivec, jv])
                                       * wi[j // _L][j % _L])
            z = out_v[pl.ds(off, _L)] + ((a[0] + a[1]) + (a[2] + a[3]))
            out_v[pl.ds(off, _L)] = 1.0 / (1.0 + jnp.exp(-z))
            return carry
        return body

    item_pass = make_item_pass(0)
    for k in range(_NK):
        cps[k].wait()
        lax.fori_loop(k * (_IDXC // _L), (k + 1) * (_IDXC // _L),
                      item_pass, jnp.int32(0))

    pltpu.sync_copy(out_v, out_hbm.at[pl.ds(base, _BPW)])


@functools.cache
def _get_rest_call():
  return pl.kernel(
    _rest_body,
    out_type=jax.ShapeDtypeStruct((_B,), jnp.float32),
    mesh=plsc.VectorSubcoreMesh(core_axis_name="c", subcore_axis_name="s",
                                num_cores=_NC, num_subcores=_NS),
    compiler_params=pltpu.CompilerParams(needs_layout_passes=False,
                                         use_tc_tiling_on_sc=False),
    scratch_types=[
        pltpu.VMEM((_NK, _IDXC), jnp.int32),
        pltpu.VMEM((_BPW,), jnp.int32),
        pltpu.VMEM((_BPW * _NINFO,), jnp.float32),
        pltpu.VMEM((_BPW,), jnp.float32),
        pltpu.VMEM((_BPW,), jnp.float32),
        pltpu.VMEM((_BPW, _F), jnp.float32),
        pltpu.VMEM((_L, _F), jnp.float32),
        pltpu.VMEM((_L,), jnp.float32),
        pltpu.VMEM((_WBLEN,), jnp.float32),
        pltpu.VMEM((_BPW,), jnp.float32),
        pltpu.VMEM((_BPW,), jnp.float32),
        pltpu.SemaphoreType.DMA((_NK,)),
    ],
  )


@jax.jit
def kernel(user_ids, shift_ids, item_ids, category, info, visits, buys,
           user_table, item_table, shift_table, W, b):
    del category
    uid = user_ids.astype(jnp.int32)
    iid = item_ids.astype(jnp.int32).reshape(_B // _IDXC, _IDXC)
    sid = shift_ids.astype(jnp.int32)
    wb = jnp.concatenate([W.reshape(-1), b.astype(jnp.float32),
                          jnp.zeros((_WBLEN - _OFF_B - 1,), jnp.float32)])
    wu_vec = W.reshape(-1)[_OFF_U:_OFF_U + _F]
    st_pad = jnp.zeros((_L, _F), jnp.float32).at[:10].set(shift_table)
    info_flat = info.reshape(-1)
    upart = _get_user_call()(uid, user_table, wu_vec)
    out = _get_rest_call()(iid, sid, info_flat, visits, buys,
                           item_table, st_pad, wb, upart)
    return out.reshape(_B, 1)


# speedup vs baseline: 1.6173x; 1.0014x over previous
import functools

import jax
import jax.numpy as jnp
from jax import lax
from jax.experimental import pallas as pl
from jax.experimental.pallas import tpu as pltpu
from jax.experimental.pallas import tpu_sc as plsc

_B = 16384
_F = 64
_NC = 2
_NS = 16
_NW = _NC * _NS
_BPW = _B // _NW
_L = 16
_HB = _BPW // 2
_HT = _HB // _L
_IDXC = 128
_NK = _BPW // _IDXC
_NINFO = 22

_OFF_SH = 24
_OFF_U = 88
_OFF_I = 152
_OFF_B = 216
_WBLEN = 240


_NSEM = 8


def _fire_row_dmas(table_hbm, ids_v, id_off, dst_v, sems):
    def grp(g, carry):
        idv = ids_v[pl.ds(id_off + g * _L, _L)]
        for k in range(_L):
            pltpu.async_copy(
                table_hbm.at[pl.ds(idv[k], 1)],
                dst_v.at[pl.ds(g * _L + k, 1)],
                sems.at[k % _NSEM])
        return carry
    lax.fori_loop(0, _HB // _L, grp, jnp.int32(0))


def _user_body(uid_hbm, ut_hbm, wu_hbm, out_hbm,
               uid_v, rowsA_v, rowsB_v, wv_v, out_v, sems_a, sems_b):
    wid = lax.axis_index("s") * _NC + lax.axis_index("c")
    base = wid * _BPW

    pltpu.sync_copy(uid_hbm.at[pl.ds(base, _BPW)], uid_v)
    _fire_row_dmas(ut_hbm, uid_v, 0, rowsA_v, sems_a)
    _fire_row_dmas(ut_hbm, uid_v, _HB, rowsB_v, sems_b)
    pltpu.sync_copy(wu_hbm, wv_v)
    w = [wv_v[pl.ds(c * _L, _L)] for c in range(_F // _L)]

    def make_pass(rows_v, t_off):
        def body(t, carry):
            ivec = t * _L + lax.iota(jnp.int32, _L)
            a = [jnp.zeros((_L,), jnp.float32) for _ in range(4)]
            for j in range(_F):
                jv = jnp.full((_L,), j, jnp.int32)
                a[j % 4] = a[j % 4] + (plsc.load_gather(rows_v, [ivec, jv])
                                       * w[j // _L][j % _L])
            out_v[pl.ds(t_off + t * _L, _L)] = (a[0] + a[1]) + (a[2] + a[3])
            return carry
        return body

    def drain(dst_v, sems):
        w = _HB // _NSEM
        for s in range(_NSEM):
            pltpu.make_async_copy(ut_hbm.at[pl.ds(0, w)],
                                  dst_v.at[pl.ds(s * w, w)], sems.at[s]).wait()

    drain(rowsA_v, sems_a)
    lax.fori_loop(0, _HT, make_pass(rowsA_v, 0), jnp.int32(0))
    drain(rowsB_v, sems_b)
    lax.fori_loop(0, _HT, make_pass(rowsB_v, _HB), jnp.int32(0))

    pltpu.sync_copy(out_v, out_hbm.at[pl.ds(base, _BPW)])


@functools.cache
def _get_user_call():
  return pl.kernel(
    _user_body,
    out_type=jax.ShapeDtypeStruct((_B,), jnp.float32),
    mesh=plsc.VectorSubcoreMesh(core_axis_name="c", subcore_axis_name="s",
                                num_cores=_NC, num_subcores=_NS),
    compiler_params=pltpu.CompilerParams(needs_layout_passes=False,
                                         use_tc_tiling_on_sc=True),
    scratch_types=[
        pltpu.VMEM((_BPW,), jnp.int32),
        pltpu.VMEM((_HB, _F), jnp.float32),
        pltpu.VMEM((_HB, _F), jnp.float32),
        pltpu.VMEM((_F,), jnp.float32),
        pltpu.VMEM((_BPW,), jnp.float32),
        pltpu.SemaphoreType.DMA((_NSEM,)),
        pltpu.SemaphoreType.DMA((_NSEM,)),
    ],
  )


def _rest_body(iid_hbm, sid_hbm, info_hbm, vis_hbm, buy_hbm,
               it_hbm, st_hbm, wb_hbm, up_hbm, out_hbm,
               iid_v, sid_v, info_v, vis_v, buy_v,
               irows_v, stab_v, sprj_v, wb_v, up_v, out_v, sems):
    wid = lax.axis_index("s") * _NC + lax.axis_index("c")
    base = wid * _BPW

    pltpu.sync_copy(iid_hbm.at[pl.ds(wid * _NK, _NK)], iid_v)
    cps = []
    for k in range(_NK):
        cps.append(pltpu.async_copy(
            it_hbm.at[iid_v.at[k]],
            irows_v.at[pl.ds(k * _IDXC, _IDXC)],
            sems.at[k]))

    pltpu.sync_copy(sid_hbm.at[pl.ds(base, _BPW)], sid_v)
    pltpu.sync_copy(info_hbm.at[pl.ds(base * _NINFO, _BPW * _NINFO)], info_v)
    pltpu.sync_copy(vis_hbm.at[pl.ds(base, _BPW)], vis_v)
    pltpu.sync_copy(buy_hbm.at[pl.ds(base, _BPW)], buy_v)
    pltpu.sync_copy(up_hbm.at[pl.ds(base, _BPW)], up_v)
    pltpu.sync_copy(wb_hbm, wb_v)
    pltpu.sync_copy(st_hbm, stab_v)

    wctx = [wb_v[pl.ds(0, _L)], wb_v[pl.ds(_L, _L)]]
    wsh = [wb_v[pl.ds(_OFF_SH + c * _L, _L)] for c in range(_F // _L)]
    wi = [wb_v[pl.ds(_OFF_I + c * _L, _L)] for c in range(_F // _L)]
    bias = wb_v[pl.ds(_OFF_B, _L)][0]

    lane = lax.iota(jnp.int32, _L)
    sprj = jnp.zeros((_L,), jnp.float32)
    for j in range(_F):
        jv = jnp.full((_L,), j, jnp.int32)
        sprj = sprj + plsc.load_gather(stab_v, [lane, jv]) * wsh[j // _L][j % _L]
    sprj_v[...] = sprj

    def ctx_body(t, carry):
        off = t * _L
        ibase = (off + lax.iota(jnp.int32, _L)) * _NINFO
        a0 = up_v[pl.ds(off, _L)] + bias
        a1 = vis_v[pl.ds(off, _L)] * wctx[1][6]
        a2 = buy_v[pl.ds(off, _L)] * wctx[1][7]
        a3 = plsc.load_gather(sprj_v, [sid_v[pl.ds(off, _L)]])
        accs = [a0, a1, a2, a3]
        for j in range(_NINFO):
            accs[j % 4] = accs[j % 4] + (plsc.load_gather(info_v, [ibase + j])
                                         * wctx[j // _L][j % _L])
        out_v[pl.ds(off, _L)] = (accs[0] + accs[1]) + (accs[2] + accs[3])
        return carry

    lax.fori_loop(0, _BPW // _L, ctx_body, jnp.int32(0))

    def make_item_pass(t0):
        def body(t, carry):
            off = t * _L
            ivec = off + lax.iota(jnp.int32, _L)
            a = [jnp.zeros((_L,), jnp.float32) for _ in range(4)]
            for j in range(_F):
                jv = jnp.full((_L,), j, jnp.int32)
                a[j % 4] = a[j % 4] + (plsc.load_gather(irows_v, [ivec, jv])
                                       * wi[j // _L][j % _L])
            z = out_v[pl.ds(off, _L)] + ((a[0] + a[1]) + (a[2] + a[3]))
            out_v[pl.ds(off, _L)] = 1.0 / (1.0 + jnp.exp(-z))
            return carry
        return body

    item_pass = make_item_pass(0)
    for k in range(_NK):
        cps[k].wait()
        lax.fori_loop(k * (_IDXC // _L), (k + 1) * (_IDXC // _L),
                      item_pass, jnp.int32(0))

    pltpu.sync_copy(out_v, out_hbm.at[pl.ds(base, _BPW)])


@functools.cache
def _get_rest_call():
  return pl.kernel(
    _rest_body,
    out_type=jax.ShapeDtypeStruct((_B,), jnp.float32),
    mesh=plsc.VectorSubcoreMesh(core_axis_name="c", subcore_axis_name="s",
                                num_cores=_NC, num_subcores=_NS),
    compiler_params=pltpu.CompilerParams(needs_layout_passes=False,
                                         use_tc_tiling_on_sc=False),
    scratch_types=[
        pltpu.VMEM((_NK, _IDXC), jnp.int32),
        pltpu.VMEM((_BPW,), jnp.int32),
        pltpu.VMEM((_BPW * _NINFO,), jnp.float32),
        pltpu.VMEM((_BPW,), jnp.float32),
        pltpu.VMEM((_BPW,), jnp.float32),
        pltpu.VMEM((_BPW, _F), jnp.float32),
        pltpu.VMEM((_L, _F), jnp.float32),
        pltpu.VMEM((_L,), jnp.float32),
        pltpu.VMEM((_WBLEN,), jnp.float32),
        pltpu.VMEM((_BPW,), jnp.float32),
        pltpu.VMEM((_BPW,), jnp.float32),
        pltpu.SemaphoreType.DMA((_NK,)),
    ],
  )


@jax.jit
def kernel(user_ids, shift_ids, item_ids, category, info, visits, buys,
           user_table, item_table, shift_table, W, b):
    del category
    uid = user_ids.astype(jnp.int32)
    iid = item_ids.astype(jnp.int32).reshape(_B // _IDXC, _IDXC)
    sid = shift_ids.astype(jnp.int32)
    wb = jnp.concatenate([W.reshape(-1), b.astype(jnp.float32),
                          jnp.zeros((_WBLEN - _OFF_B - 1,), jnp.float32)])
    wu_vec = W.reshape(-1)[_OFF_U:_OFF_U + _F]
    st_pad = jnp.zeros((_L, _F), jnp.float32).at[:10].set(shift_table)
    info_flat = info.reshape(-1)
    upart = _get_user_call()(uid, user_table, wu_vec)
    out = _get_rest_call()(iid, sid, info_flat, visits, buys,
                           item_table, st_pad, wb, upart)
    return out.reshape(_B, 1)
